# baseline (speedup 1.0000x reference)
import functools

import jax
import jax.numpy as jnp
from jax import lax
from jax.experimental import pallas as pl
from jax.experimental.pallas import tpu as pltpu
from jax.experimental.pallas import tpu_sc as plsc

_B = 16384
_HID = 4096
_NPL = 1000
_TB = 2048
_TH = 1024
_NW = 32
_BPW = _B // _NW


def _sc_gather_body(embflat_hbm, ids_hbm, out_hbm, ids_v, idx_v, row_v, sem):
    wid = lax.axis_index("s") * 2 + lax.axis_index("c")
    base = wid * _BPW
    pltpu.sync_copy(ids_hbm.at[pl.ds(base, _BPW)], ids_v)
    for d in range(8):
        for j in range(_BPW // 16):
            sl = pl.ds(j * 16, 16)
            idx_v[sl] = ids_v[sl] + d * _NPL
        pltpu.async_copy(embflat_hbm.at[idx_v], row_v, sem).wait()
        pltpu.sync_copy(row_v, out_hbm.at[d, pl.ds(base, _BPW)])


def _tc_body(xpt_ref, embt_ref, wtt_ref, wht_ref, out_ref):
    xf = xpt_ref[...]
    state = jnp.concatenate(
        [xf[:18].astype(jnp.bfloat16), embt_ref[...].astype(jnp.bfloat16)],
        axis=0)
    logits = jnp.zeros((8, _TB), jnp.float32)
    for c in range(_HID // _TH):
        ht = jnp.dot(wtt_ref[c * _TH:(c + 1) * _TH, :], state,
                     preferred_element_type=jnp.float32)
        hb = jnp.maximum(ht.astype(jnp.bfloat16), jnp.bfloat16(0))
        logits = logits + jnp.dot(wht_ref[:, c * _TH:(c + 1) * _TH], hb,
                                  preferred_element_type=jnp.float32)
    out_ref[...] = logits


@jax.jit
def kernel(x, W_fc, b_fc, emb, W_land, W_shot, W_move):
    x = x.astype(jnp.float32)
    ids = x[:, 17].astype(jnp.int32)
    embflat = emb.T.reshape(8 * _NPL).astype(jnp.float32)

    sc_gather = functools.partial(
        pl.kernel,
        mesh=plsc.VectorSubcoreMesh(core_axis_name="c", subcore_axis_name="s"),
        out_type=jax.ShapeDtypeStruct((8, _B), jnp.float32),
        scratch_types=[
            pltpu.VMEM((_BPW,), jnp.int32),
            pltpu.VMEM((_BPW,), jnp.int32),
            pltpu.VMEM((_BPW,), jnp.float32),
            pltpu.SemaphoreType.DMA,
        ],
    )(_sc_gather_body)
    embt_all = sc_gather(embflat, ids)

    xpt = jnp.concatenate(
        [x[:, :17].T, jnp.ones((1, _B), jnp.float32),
         jnp.zeros((14, _B), jnp.float32)], axis=0)
    wtt = jnp.concatenate(
        [W_fc[:, :17], b_fc[:, None], W_fc[:, 17:]],
        axis=1).astype(jnp.bfloat16)
    wht = jnp.concatenate(
        [W_land, W_shot, W_move, jnp.zeros((1, _HID), jnp.float32)],
        axis=0).astype(jnp.bfloat16)

    grid = (_B // _TB,)
    outT = pl.pallas_call(
        _tc_body,
        grid=grid,
        in_specs=[
            pl.BlockSpec((32, _TB), lambda i: (0, i)),
            pl.BlockSpec((8, _TB), lambda i: (0, i)),
            pl.BlockSpec((_HID, 26), lambda i: (0, 0)),
            pl.BlockSpec((8, _HID), lambda i: (0, 0)),
        ],
        out_specs=pl.BlockSpec((8, _TB), lambda i: (0, i)),
        out_shape=jax.ShapeDtypeStruct((8, _B), jnp.float32),
    )(xpt, embt_all, wtt, wht)
    return (outT[0:2].T, outT[2:5].T, outT[5:7].T)


# speedup vs baseline: 4.8056x; 4.8056x over previous
import jax
import jax.numpy as jnp
from jax import lax
from jax.experimental import pallas as pl

_B = 16384
_HID = 4096
_NPL = 1000
_TB = 4096
_TH = 1024


def _fused_body(xpt_ref, wtt_ref, embt_ref, wht_ref, out_ref):
    xf = xpt_ref[...]
    ids = xf[18:19, :].astype(jnp.int32)
    iota = lax.broadcasted_iota(jnp.int32, (_NPL, _TB), 0)
    onehot = (ids == iota).astype(jnp.bfloat16)
    embeds = jnp.dot(embt_ref[...], onehot,
                     preferred_element_type=jnp.float32)
    state = jnp.concatenate(
        [xf[:18].astype(jnp.bfloat16), embeds.astype(jnp.bfloat16)],
        axis=0)
    logits = jnp.zeros((8, _TB), jnp.float32)
    for c in range(_HID // _TH):
        ht = jnp.dot(wtt_ref[c * _TH:(c + 1) * _TH, :], state,
                     preferred_element_type=jnp.float32)
        hb = jnp.maximum(ht.astype(jnp.bfloat16), jnp.bfloat16(0))
        logits = logits + jnp.dot(wht_ref[:, c * _TH:(c + 1) * _TH], hb,
                                  preferred_element_type=jnp.float32)
    out_ref[...] = logits


@jax.jit
def kernel(x, W_fc, b_fc, emb, W_land, W_shot, W_move):
    x = x.astype(jnp.float32)
    xpt = jnp.concatenate(
        [x[:, :17].T, jnp.ones((1, _B), jnp.float32), x[:, 17:18].T,
         jnp.zeros((13, _B), jnp.float32)], axis=0)
    wtt = jnp.concatenate(
        [W_fc[:, :17], b_fc[:, None], W_fc[:, 17:]],
        axis=1).astype(jnp.bfloat16)
    wht = jnp.concatenate(
        [W_land, W_shot, W_move, jnp.zeros((1, _HID), jnp.float32)],
        axis=0).astype(jnp.bfloat16)
    embt = emb.T.astype(jnp.bfloat16)

    grid = (_B // _TB,)
    outT = pl.pallas_call(
        _fused_body,
        grid=grid,
        in_specs=[
            pl.BlockSpec((32, _TB), lambda i: (0, i)),
            pl.BlockSpec((_HID, 26), lambda i: (0, 0)),
            pl.BlockSpec((8, _NPL), lambda i: (0, 0)),
            pl.BlockSpec((8, _HID), lambda i: (0, 0)),
        ],
        out_specs=pl.BlockSpec((8, _TB), lambda i: (0, i)),
        out_shape=jax.ShapeDtypeStruct((8, _B), jnp.float32),
    )(xpt, wtt, embt, wht)
    return (outT[0:2].T, outT[2:5].T, outT[5:7].T)
